# baseline (speedup 1.0000x reference)
import dataclasses
import functools

import jax
import jax.numpy as jnp
from jax import lax
from jax.experimental import pallas as pl
from jax.experimental.pallas import tpu as pltpu
from jax.experimental.pallas import tpu_sc as plsc

B = 16384
N_NUM = 13
F = 26
V = 100000
VPAD = 100032
K = 16
NC = 2
NS = 16
NW = NC * NS
ROWS_W = B // NW
RC = 64
NCHUNK = ROWS_W // RC
RU = 8
FP = 32


def _repack(v_cat):
    vt = jnp.transpose(v_cat, (0, 2, 1))
    CH = 1024
    NCH = V // CH
    TAIL = V - NCH * CH

    def body(in_ref, out_ref):
        def pack(w, n):
            y3 = jnp.transpose(w).reshape(n // 8, 8, K)
            return jnp.concatenate([y3[:, s, :] for s in range(8)], axis=1)

        @pl.loop(0, NCH)
        def _(c):
            w = in_ref[0, :, pl.ds(c * CH, CH)]
            out_ref[pl.ds(c * (CH // 8), CH // 8), :] = pack(w, CH)
        w = in_ref[0, :, pl.ds(NCH * CH, TAIL)]
        out_ref[pl.ds(NCH * (CH // 8), TAIL // 8), :] = pack(w, TAIL)

    return pl.pallas_call(
        body,
        grid=(F,),
        in_specs=[pl.BlockSpec((1, K, V), lambda f: (f, 0, 0))],
        out_specs=pl.BlockSpec((VPAD // 8, 8 * K), lambda f: (f, 0)),
        out_shape=jax.ShapeDtypeStruct((F * VPAD // 8, 8 * K), jnp.float32),
    )(vt)


def _sc_gather(vflat, lflat, idxv, idxl):
    mesh = plsc.VectorSubcoreMesh(core_axis_name="c", subcore_axis_name="s")
    cp = pltpu.CompilerParams()
    if "use_tc_tiling_on_sc" in pltpu.CompilerParams.__dataclass_fields__:
        cp = dataclasses.replace(cp, use_tc_tiling_on_sc=False)

    @functools.partial(
        pl.kernel,
        out_type=jax.ShapeDtypeStruct((B, 3 * K), jnp.float32),
        mesh=mesh,
        compiler_params=cp,
        scratch_types=[
            pltpu.VMEM((ROWS_W * F,), jnp.int32),
            pltpu.VMEM((ROWS_W * FP,), jnp.int32),
            pltpu.VMEM((RC * F, K), jnp.float32),
            pltpu.VMEM((RC * F, K), jnp.float32),
            pltpu.VMEM((RC * FP,), jnp.float32),
            pltpu.VMEM((RC * FP,), jnp.float32),
            pltpu.VMEM((ROWS_W, 3 * K), jnp.float32),
            pltpu.SemaphoreType.DMA,
            pltpu.SemaphoreType.DMA,
            pltpu.SemaphoreType.DMA,
            pltpu.SemaphoreType.DMA,
        ],
    )
    def k(vflat_hbm, lflat_hbm, idxv_hbm, idxl_hbm, out_hbm,
          idxv_v, idxl_v, vrows_a, vrows_b, lrows_a, lrows_b, outbuf,
          sem_va, sem_vb, sem_la, sem_lb):
        wid = lax.axis_index("s") * NC + lax.axis_index("c")
        base = wid * ROWS_W
        RCF = RC * F
        RCL = RC * FP
        pltpu.sync_copy(idxv_hbm.at[pl.ds(base * F, ROWS_W * F)], idxv_v)
        pltpu.sync_copy(idxl_hbm.at[pl.ds(base * FP, ROWS_W * FP)], idxl_v)

        def issue(c, vr, lr, sv, sl):
            pltpu.async_copy(
                vflat_hbm.at[idxv_v.at[pl.ds(c * RCF, RCF)]], vr, sv)
            pltpu.async_copy(
                lflat_hbm.at[idxl_v.at[pl.ds(c * RCL, RCL)]], lr, sl)

        def wait(c, vr, lr, sv, sl):
            pltpu.make_async_copy(
                vflat_hbm.at[idxv_v.at[pl.ds(c * RCF, RCF)]], vr, sv).wait()
            pltpu.make_async_copy(
                lflat_hbm.at[idxl_v.at[pl.ds(c * RCL, RCL)]], lr, sl).wait()

        def compute(c, vr, lr):
            @pl.loop(0, RC, step=RU)
            def _(r0):
                for u in range(RU):
                    r = r0 + u
                    acc = vr[r * F]
                    acc2 = acc * acc
                    for j in range(1, F):
                        e = vr[r * F + j]
                        acc = acc + e
                        acc2 = acc2 + e * e
                    lp = lr[pl.ds(r * FP, K)] + lr[pl.ds(r * FP + K, K)]
                    row = c * RC + r
                    outbuf[row, pl.ds(0, K)] = acc
                    outbuf[row, pl.ds(K, K)] = acc2
                    outbuf[row, pl.ds(2 * K, K)] = lp

        issue(0, vrows_a, lrows_a, sem_va, sem_la)

        @pl.loop(0, NCHUNK, step=2)
        def _(c):
            issue(c + 1, vrows_b, lrows_b, sem_vb, sem_lb)
            wait(c, vrows_a, lrows_a, sem_va, sem_la)
            compute(c, vrows_a, lrows_a)

            @pl.when(c + 2 < NCHUNK)
            def _():
                issue(c + 2, vrows_a, lrows_a, sem_va, sem_la)

            wait(c + 1, vrows_b, lrows_b, sem_vb, sem_lb)
            compute(c + 1, vrows_b, lrows_b)

        pltpu.sync_copy(outbuf, out_hbm.at[pl.ds(base, ROWS_W)])

    return k(vflat, lflat, idxv, idxl)


def _combine(scout, x_num, v_num, w_row, const):
    BLK = 512

    def body(sc_ref, x_ref, vn_ref, w_ref, c_ref, o_ref):
        sc = sc_ref[...]
        x = x_ref[...]
        vn = vn_ref[...]
        sv = sc[:, 0:K] + jnp.dot(x, vn, preferred_element_type=jnp.float32)
        sq = sc[:, K:2 * K] + jnp.dot(x * x, vn * vn,
                                      preferred_element_type=jnp.float32)
        lin = (jnp.sum(sc[:, 2 * K:3 * K], axis=1, keepdims=True)
               + jnp.sum(x * w_ref[...], axis=1, keepdims=True)
               + c_ref[0, 0])
        o_ref[...] = lin + 0.5 * jnp.sum(sv * sv - sq, axis=1, keepdims=True)

    return pl.pallas_call(
        body,
        grid=(B // BLK,),
        in_specs=[
            pl.BlockSpec((BLK, 3 * K), lambda i: (i, 0)),
            pl.BlockSpec((BLK, N_NUM), lambda i: (i, 0)),
            pl.BlockSpec((N_NUM, K), lambda i: (0, 0)),
            pl.BlockSpec((1, N_NUM), lambda i: (0, 0)),
            pl.BlockSpec((1, 1), lambda i: (0, 0)),
        ],
        out_specs=pl.BlockSpec((BLK, 1), lambda i: (i, 0)),
        out_shape=jax.ShapeDtypeStruct((B, 1), jnp.float32),
    )(scout, x_num, v_num, w_row, const)


def kernel(x_num, x_cat, bias, W_num, lin_cat, v_num, v_cat):
    xc = x_cat.astype(jnp.int32)
    idxv = (xc + jnp.arange(F, dtype=jnp.int32)[None, :] * VPAD).reshape(-1)
    idxl2 = xc + jnp.arange(F, dtype=jnp.int32)[None, :] * V
    idxl = jnp.concatenate(
        [idxl2, jnp.zeros((B, FP - F), jnp.int32)], axis=1).reshape(-1)
    vflat = _repack(v_cat).reshape(F * VPAD, K)
    lflat = lin_cat.reshape(F * V)
    scout = _sc_gather(vflat, lflat, idxv, idxl)
    const = (bias[0] - (FP - F) * lin_cat[0, 0, 0]).reshape(1, 1)
    return _combine(scout, x_num, v_num, W_num, const)


# speedup vs baseline: 2.5925x; 2.5925x over previous
import dataclasses
import functools

import jax
import jax.numpy as jnp
from jax import lax
from jax.experimental import pallas as pl
from jax.experimental.pallas import tpu as pltpu
from jax.experimental.pallas import tpu_sc as plsc

B = 16384
N_NUM = 13
F = 26
V = 100000
VP = 100096
K = 16
NC = 2
NS = 16
BH = B // NC


def _detile(v_cat):
    vt = jnp.transpose(v_cat, (0, 2, 1))

    def body(in_ref, out_ref):
        x = in_ref[0]
        out_ref[...] = jnp.concatenate(
            [x, jnp.zeros((K, VP - V), jnp.float32)], axis=1)

    return pl.pallas_call(
        body,
        grid=(F,),
        in_specs=[pl.BlockSpec((1, K, V), lambda f: (f, 0, 0))],
        out_specs=pl.BlockSpec((K, VP), lambda f: (f, 0)),
        out_shape=jax.ShapeDtypeStruct((F * K, VP), jnp.float32),
    )(vt)


def _sc_planes(vplanes, lflat, idxt):
    mesh = plsc.VectorSubcoreMesh(core_axis_name="c", subcore_axis_name="s")
    cp = pltpu.CompilerParams()
    if "use_tc_tiling_on_sc" in pltpu.CompilerParams.__dataclass_fields__:
        cp = dataclasses.replace(cp, use_tc_tiling_on_sc=False)
    if "needs_layout_passes" in pltpu.CompilerParams.__dataclass_fields__:
        cp = dataclasses.replace(cp, needs_layout_passes=False)

    @functools.partial(
        pl.kernel,
        out_type=jax.ShapeDtypeStruct((NC, 3, NS, BH), jnp.float32),
        mesh=mesh,
        compiler_params=cp,
        scratch_types=[
            pltpu.VMEM((V,), jnp.float32),
            pltpu.VMEM((BH,), jnp.int32),
            pltpu.VMEM((BH,), jnp.float32),
            pltpu.VMEM((BH,), jnp.float32),
            pltpu.SemaphoreType.DMA,
        ],
    )
    def k(vp_hbm, lf_hbm, idx_hbm, out_hbm, plane, idxf, acc, acc2, sem):
        c = lax.axis_index("c")
        t = lax.axis_index("s")
        bbase = c * BH

        @pl.loop(0, BH, step=16)
        def _(m):
            z = jnp.zeros((16,), jnp.float32)
            acc[pl.ds(m, 16)] = z
            acc2[pl.ds(m, 16)] = z

        @pl.loop(0, F)
        def _(f):
            pltpu.async_copy(
                vp_hbm.at[pl.ds((f * K + t) * VP, V)], plane, sem).wait()
            pltpu.async_copy(
                idx_hbm.at[pl.ds(f * B + bbase, BH)], idxf, sem).wait()

            @pl.loop(0, BH, step=16)
            def _(m):
                g = plsc.load_gather(plane, [idxf[pl.ds(m, 16)]])
                acc[pl.ds(m, 16)] = acc[pl.ds(m, 16)] + g
                acc2[pl.ds(m, 16)] = acc2[pl.ds(m, 16)] + g * g

        pltpu.sync_copy(acc, out_hbm.at[c, 0, t])
        pltpu.sync_copy(acc2, out_hbm.at[c, 1, t])

        @pl.loop(0, BH, step=16)
        def _(m):
            acc[pl.ds(m, 16)] = jnp.zeros((16,), jnp.float32)

        def lin_field(f):
            pltpu.async_copy(
                lf_hbm.at[pl.ds(f * V, V)], plane, sem).wait()
            pltpu.async_copy(
                idx_hbm.at[pl.ds(f * B + bbase, BH)], idxf, sem).wait()

            @pl.loop(0, BH, step=16)
            def _(m):
                g = plsc.load_gather(plane, [idxf[pl.ds(m, 16)]])
                acc[pl.ds(m, 16)] = acc[pl.ds(m, 16)] + g

        lin_field(t)

        @pl.when(t + NS < F)
        def _():
            lin_field(t + NS)

        pltpu.sync_copy(acc, out_hbm.at[c, 2, t])

    return k(vplanes, lflat, idxt)


def _combine(scout, x_num, v_num, w_row, const):
    BLK = 512
    NBH = BH // BLK

    def body(sc_ref, x_ref, vn_ref, w_ref, c_ref, o_ref):
        sc = sc_ref[0]
        sv_cat = jnp.transpose(sc[0])
        sq_cat = jnp.transpose(sc[1])
        lp = jnp.transpose(sc[2])
        x = x_ref[...]
        vn = vn_ref[...]
        sv = sv_cat + jnp.dot(x, vn, preferred_element_type=jnp.float32)
        sq = sq_cat + jnp.dot(x * x, vn * vn,
                              preferred_element_type=jnp.float32)
        lin = (jnp.sum(lp, axis=1, keepdims=True)
               + jnp.sum(x * w_ref[...], axis=1, keepdims=True)
               + c_ref[0, 0])
        o_ref[...] = lin + 0.5 * jnp.sum(sv * sv - sq, axis=1, keepdims=True)

    return pl.pallas_call(
        body,
        grid=(B // BLK,),
        in_specs=[
            pl.BlockSpec((1, 3, NS, BLK), lambda i: (i // NBH, 0, 0, i % NBH)),
            pl.BlockSpec((BLK, N_NUM), lambda i: (i, 0)),
            pl.BlockSpec((N_NUM, K), lambda i: (0, 0)),
            pl.BlockSpec((1, N_NUM), lambda i: (0, 0)),
            pl.BlockSpec((1, 1), lambda i: (0, 0)),
        ],
        out_specs=pl.BlockSpec((BLK, 1), lambda i: (i, 0)),
        out_shape=jax.ShapeDtypeStruct((B, 1), jnp.float32),
    )(scout, x_num, v_num, w_row, const)


def kernel(x_num, x_cat, bias, W_num, lin_cat, v_num, v_cat):
    xc = x_cat.astype(jnp.int32)
    idxt = jnp.transpose(xc).reshape(-1)
    vplanes = _detile(v_cat).reshape(-1)
    lflat = lin_cat.reshape(F * V)
    scout = _sc_planes(vplanes, lflat, idxt)
    const = bias.reshape(1, 1)
    return _combine(scout, x_num, v_num, W_num, const)


# speedup vs baseline: 5.2511x; 2.0255x over previous
import dataclasses
import functools

import jax
import jax.numpy as jnp
from jax import lax
from jax.experimental import pallas as pl
from jax.experimental.pallas import tpu as pltpu
from jax.experimental.pallas import tpu_sc as plsc

B = 16384
N_NUM = 13
F = 26
V = 100000
VP = 100096
K = 16
NC = 2
NS = 16
BH = B // NC


def _detile(v_cat):
    vt = jnp.transpose(v_cat, (0, 2, 1))

    def body(in_ref, out_ref):
        x = in_ref[0]
        out_ref[...] = jnp.concatenate(
            [x, jnp.zeros((K, VP - V), jnp.float32)], axis=1)

    return pl.pallas_call(
        body,
        grid=(F,),
        in_specs=[pl.BlockSpec((1, K, V), lambda f: (f, 0, 0))],
        out_specs=pl.BlockSpec((K, VP), lambda f: (f, 0)),
        out_shape=jax.ShapeDtypeStruct((F * K, VP), jnp.float32),
    )(vt)


def _sc_planes(vt3, lint3, idxt):
    mesh = plsc.VectorSubcoreMesh(core_axis_name="c", subcore_axis_name="s")
    cp = pltpu.CompilerParams()
    if "use_tc_tiling_on_sc" in pltpu.CompilerParams.__dataclass_fields__:
        cp = dataclasses.replace(cp, use_tc_tiling_on_sc=True)
    if "needs_layout_passes" in pltpu.CompilerParams.__dataclass_fields__:
        cp = dataclasses.replace(cp, needs_layout_passes=False)

    @functools.partial(
        pl.kernel,
        out_type=jax.ShapeDtypeStruct((NC * 3 * NS * BH,), jnp.float32),
        mesh=mesh,
        compiler_params=cp,
        scratch_types=[
            pltpu.VMEM((V,), jnp.float32),
            pltpu.VMEM((BH,), jnp.int32),
            pltpu.VMEM((BH,), jnp.float32),
            pltpu.VMEM((BH,), jnp.float32),
            pltpu.SemaphoreType.DMA,
        ],
    )
    def k(vt_hbm, lf_hbm, idx_hbm, out_hbm, plane, idxf, acc, acc2, sem):
        c = lax.axis_index("c")
        t = lax.axis_index("s")
        bbase = c * BH

        @pl.loop(0, BH, step=16)
        def _(m):
            z = jnp.zeros((16,), jnp.float32)
            acc[pl.ds(m, 16)] = z
            acc2[pl.ds(m, 16)] = z

        @pl.loop(0, F)
        def _(f):
            pltpu.async_copy(vt_hbm.at[f, t, :], plane, sem).wait()
            pltpu.async_copy(
                idx_hbm.at[pl.ds(f * B + bbase, BH)], idxf, sem).wait()

            @pl.loop(0, BH, step=16)
            def _(m):
                g = plsc.load_gather(plane, [idxf[pl.ds(m, 16)]])
                acc[pl.ds(m, 16)] = acc[pl.ds(m, 16)] + g
                acc2[pl.ds(m, 16)] = acc2[pl.ds(m, 16)] + g * g

        pltpu.sync_copy(acc, out_hbm.at[pl.ds(((c * 3 + 0) * NS + t) * BH, BH)])
        pltpu.sync_copy(acc2, out_hbm.at[pl.ds(((c * 3 + 1) * NS + t) * BH, BH)])

        @pl.loop(0, BH, step=16)
        def _(m):
            acc[pl.ds(m, 16)] = jnp.zeros((16,), jnp.float32)

        def lin_field(f):
            pltpu.async_copy(lf_hbm.at[f, 0, :], plane, sem).wait()
            pltpu.async_copy(
                idx_hbm.at[pl.ds(f * B + bbase, BH)], idxf, sem).wait()

            @pl.loop(0, BH, step=16)
            def _(m):
                g = plsc.load_gather(plane, [idxf[pl.ds(m, 16)]])
                acc[pl.ds(m, 16)] = acc[pl.ds(m, 16)] + g

        lin_field(t)

        @pl.when(t + NS < F)
        def _():
            lin_field(t + NS)

        pltpu.sync_copy(acc, out_hbm.at[pl.ds(((c * 3 + 2) * NS + t) * BH, BH)])

    return k(vt3, lint3, idxt)


def _combine(scout, x_num, v_num, w_row, const):
    BLK = 512
    NBH = BH // BLK

    def body(sc_ref, x_ref, vn_ref, w_ref, c_ref, o_ref):
        sc = sc_ref[0]
        sv_cat = jnp.transpose(sc[0])
        sq_cat = jnp.transpose(sc[1])
        lp = jnp.transpose(sc[2])
        x = x_ref[...]
        vn = vn_ref[...]
        sv = sv_cat + jnp.dot(x, vn, preferred_element_type=jnp.float32)
        sq = sq_cat + jnp.dot(x * x, vn * vn,
                              preferred_element_type=jnp.float32)
        lin = (jnp.sum(lp, axis=1, keepdims=True)
               + jnp.sum(x * w_ref[...], axis=1, keepdims=True)
               + c_ref[0, 0])
        o_ref[...] = lin + 0.5 * jnp.sum(sv * sv - sq, axis=1, keepdims=True)

    return pl.pallas_call(
        body,
        grid=(B // BLK,),
        in_specs=[
            pl.BlockSpec((1, 3, NS, BLK), lambda i: (i // NBH, 0, 0, i % NBH)),
            pl.BlockSpec((BLK, N_NUM), lambda i: (i, 0)),
            pl.BlockSpec((N_NUM, K), lambda i: (0, 0)),
            pl.BlockSpec((1, N_NUM), lambda i: (0, 0)),
            pl.BlockSpec((1, 1), lambda i: (0, 0)),
        ],
        out_specs=pl.BlockSpec((BLK, 1), lambda i: (i, 0)),
        out_shape=jax.ShapeDtypeStruct((B, 1), jnp.float32),
    )(scout, x_num, v_num, w_row, const)


def kernel(x_num, x_cat, bias, W_num, lin_cat, v_num, v_cat):
    xc = x_cat.astype(jnp.int32)
    idxt = jnp.transpose(xc).reshape(-1)
    vt3 = jnp.transpose(v_cat, (0, 2, 1))
    lint3 = jnp.transpose(lin_cat, (0, 2, 1))
    scout = _sc_planes(vt3, lint3, idxt).reshape(NC, 3, NS, BH)
    const = bias.reshape(1, 1)
    return _combine(scout, x_num, v_num, W_num, const)


# speedup vs baseline: 5.2972x; 1.0088x over previous
import dataclasses
import functools

import jax
import jax.numpy as jnp
from jax import lax
from jax.experimental import pallas as pl
from jax.experimental.pallas import tpu as pltpu
from jax.experimental.pallas import tpu_sc as plsc

B = 16384
N_NUM = 13
F = 26
V = 100000
VP = 100096
K = 16
NC = 2
NS = 16
BH = B // NC


def _detile(v_cat):
    vt = jnp.transpose(v_cat, (0, 2, 1))

    def body(in_ref, out_ref):
        x = in_ref[0]
        out_ref[...] = jnp.concatenate(
            [x, jnp.zeros((K, VP - V), jnp.float32)], axis=1)

    return pl.pallas_call(
        body,
        grid=(F,),
        in_specs=[pl.BlockSpec((1, K, V), lambda f: (f, 0, 0))],
        out_specs=pl.BlockSpec((K, VP), lambda f: (f, 0)),
        out_shape=jax.ShapeDtypeStruct((F * K, VP), jnp.float32),
    )(vt)


def _sc_planes(vt3, lint3, idxt):
    mesh = plsc.VectorSubcoreMesh(core_axis_name="c", subcore_axis_name="s")
    cp = pltpu.CompilerParams()
    if "use_tc_tiling_on_sc" in pltpu.CompilerParams.__dataclass_fields__:
        cp = dataclasses.replace(cp, use_tc_tiling_on_sc=True)
    if "needs_layout_passes" in pltpu.CompilerParams.__dataclass_fields__:
        cp = dataclasses.replace(cp, needs_layout_passes=False)

    @functools.partial(
        pl.kernel,
        out_type=jax.ShapeDtypeStruct((NC * 3 * NS * BH,), jnp.float32),
        mesh=mesh,
        compiler_params=cp,
        scratch_types=[
            pltpu.VMEM((V,), jnp.float32),
            pltpu.VMEM((BH,), jnp.int32),
            pltpu.VMEM((BH,), jnp.float32),
            pltpu.VMEM((BH,), jnp.float32),
            pltpu.SemaphoreType.DMA,
            pltpu.SemaphoreType.DMA,
        ],
    )
    def k(vt_hbm, lf_hbm, idx_hbm, out_hbm, plane, idxf, acc, acc2,
          sem, sem2):
        c = lax.axis_index("c")
        t = lax.axis_index("s")
        bbase = c * BH

        @pl.loop(0, BH, step=16)
        def _(m):
            z = jnp.zeros((16,), jnp.float32)
            acc[pl.ds(m, 16)] = z
            acc2[pl.ds(m, 16)] = z

        @pl.loop(0, F)
        def _(f):
            cp = pltpu.async_copy(vt_hbm.at[f, t, :], plane, sem)
            ci = pltpu.async_copy(
                idx_hbm.at[pl.ds(f * B + bbase, BH)], idxf, sem2)
            cp.wait()
            ci.wait()

            @pl.loop(0, BH, step=64)
            def _(m0):
                for u in range(4):
                    m = m0 + u * 16
                    g = plsc.load_gather(plane, [idxf[pl.ds(m, 16)]])
                    acc[pl.ds(m, 16)] = acc[pl.ds(m, 16)] + g
                    acc2[pl.ds(m, 16)] = acc2[pl.ds(m, 16)] + g * g

        pltpu.sync_copy(acc, out_hbm.at[pl.ds(((c * 3 + 0) * NS + t) * BH, BH)])
        pltpu.sync_copy(acc2, out_hbm.at[pl.ds(((c * 3 + 1) * NS + t) * BH, BH)])

        @pl.loop(0, BH, step=16)
        def _(m):
            acc[pl.ds(m, 16)] = jnp.zeros((16,), jnp.float32)

        def lin_field(f):
            cp = pltpu.async_copy(lf_hbm.at[f, 0, :], plane, sem)
            ci = pltpu.async_copy(
                idx_hbm.at[pl.ds(f * B + bbase, BH)], idxf, sem2)
            cp.wait()
            ci.wait()

            @pl.loop(0, BH, step=64)
            def _(m0):
                for u in range(4):
                    m = m0 + u * 16
                    g = plsc.load_gather(plane, [idxf[pl.ds(m, 16)]])
                    acc[pl.ds(m, 16)] = acc[pl.ds(m, 16)] + g

        lin_field(t)

        @pl.when(t + NS < F)
        def _():
            lin_field(t + NS)

        pltpu.sync_copy(acc, out_hbm.at[pl.ds(((c * 3 + 2) * NS + t) * BH, BH)])

    return k(vt3, lint3, idxt)


def _combine(scout, x_num, v_num, w_row, const):
    BLK = 512
    NBH = BH // BLK

    def body(sc_ref, x_ref, vn_ref, w_ref, c_ref, o_ref):
        sc = sc_ref[0]
        sv_cat = jnp.transpose(sc[0])
        sq_cat = jnp.transpose(sc[1])
        lp = jnp.transpose(sc[2])
        x = x_ref[...]
        vn = vn_ref[...]
        sv = sv_cat + jnp.dot(x, vn, preferred_element_type=jnp.float32)
        sq = sq_cat + jnp.dot(x * x, vn * vn,
                              preferred_element_type=jnp.float32)
        lin = (jnp.sum(lp, axis=1, keepdims=True)
               + jnp.sum(x * w_ref[...], axis=1, keepdims=True)
               + c_ref[0, 0])
        o_ref[...] = lin + 0.5 * jnp.sum(sv * sv - sq, axis=1, keepdims=True)

    return pl.pallas_call(
        body,
        grid=(B // BLK,),
        in_specs=[
            pl.BlockSpec((1, 3, NS, BLK), lambda i: (i // NBH, 0, 0, i % NBH)),
            pl.BlockSpec((BLK, N_NUM), lambda i: (i, 0)),
            pl.BlockSpec((N_NUM, K), lambda i: (0, 0)),
            pl.BlockSpec((1, N_NUM), lambda i: (0, 0)),
            pl.BlockSpec((1, 1), lambda i: (0, 0)),
        ],
        out_specs=pl.BlockSpec((BLK, 1), lambda i: (i, 0)),
        out_shape=jax.ShapeDtypeStruct((B, 1), jnp.float32),
    )(scout, x_num, v_num, w_row, const)


def kernel(x_num, x_cat, bias, W_num, lin_cat, v_num, v_cat):
    xc = x_cat.astype(jnp.int32)
    idxt = jnp.transpose(xc).reshape(-1)
    vt3 = jnp.transpose(v_cat, (0, 2, 1))
    lint3 = jnp.transpose(lin_cat, (0, 2, 1))
    scout = _sc_planes(vt3, lint3, idxt).reshape(NC, 3, NS, BH)
    const = bias.reshape(1, 1)
    return _combine(scout, x_num, v_num, W_num, const)
